# baseline (speedup 1.0000x reference)
import functools

import jax
import jax.numpy as jnp
from jax import lax
from jax.experimental import pallas as pl
from jax.experimental.pallas import tpu as pltpu
from jax.experimental.pallas import tpu_sc as plsc

HIDDEN = 64
SEQ_LEN = 200
BATCH = 4096
NC, NS = 2, 16
NW = NC * NS
BPW = BATCH // NW
HALF = SEQ_LEN // 2
LANES = 16


def _build():
    mesh = plsc.VectorSubcoreMesh(core_axis_name="c", subcore_axis_name="s")

    @functools.partial(
        pl.kernel,
        out_type=jax.ShapeDtypeStruct((BATCH, SEQ_LEN, HIDDEN), jnp.float32),
        mesh=mesh,
        scratch_types=[
            pltpu.VMEM((2, HALF), jnp.int32),
            pltpu.VMEM((SEQ_LEN, HIDDEN), jnp.float32),
            pltpu.VMEM((SEQ_LEN, HIDDEN), jnp.float32),
            pltpu.SemaphoreType.DMA,
        ],
        compiler_params=pltpu.CompilerParams(use_tc_tiling_on_sc=False),
    )
    def k(idx_hbm, table_hbm, pos_hbm, out_hbm, idx_v, buf, pos_v, sem):
        wid = lax.axis_index("s") * NC + lax.axis_index("c")
        pltpu.sync_copy(pos_hbm, pos_v)

        @pl.loop(0, BPW)
        def _seq(i):
            b = wid * BPW + i
            pltpu.sync_copy(idx_hbm.at[b], idx_v)
            c0 = pltpu.async_copy(table_hbm.at[idx_v.at[0]],
                                  buf.at[pl.ds(0, HALF)], sem)
            c1 = pltpu.async_copy(table_hbm.at[idx_v.at[1]],
                                  buf.at[pl.ds(HALF, HALF)], sem)
            c0.wait()
            c1.wait()

            @pl.loop(0, SEQ_LEN)
            def _row(r):
                for d in range(HIDDEN // LANES):
                    sl = pl.ds(d * LANES, LANES)
                    buf[r, sl] = buf[r, sl] + pos_v[r, sl]

            pltpu.sync_copy(buf, out_hbm.at[b])

    return k


_KERNEL = _build()


def kernel(item_seq, ID_embeddings, positional_embeddings):
    idx = item_seq.reshape(BATCH, 2, HALF)
    return _KERNEL(idx, ID_embeddings, positional_embeddings)


# speedup vs baseline: 1.2101x; 1.2101x over previous
import functools

import jax
import jax.numpy as jnp
from jax import lax
from jax.experimental import pallas as pl
from jax.experimental.pallas import tpu as pltpu
from jax.experimental.pallas import tpu_sc as plsc

HIDDEN = 64
SEQ_LEN = 200
BATCH = 4096
NC, NS = 2, 16
NW = NC * NS
BPW = BATCH // NW
HALF = SEQ_LEN // 2
LANES = 16
NBUF = 4


def _build():
    mesh = plsc.VectorSubcoreMesh(core_axis_name="c", subcore_axis_name="s")

    @functools.partial(
        pl.kernel,
        out_type=jax.ShapeDtypeStruct((BATCH, SEQ_LEN, HIDDEN), jnp.float32),
        mesh=mesh,
        scratch_types=[
            pltpu.VMEM((BPW, 2, HALF), jnp.int32),
            pltpu.VMEM((NBUF, SEQ_LEN, HIDDEN), jnp.float32),
            pltpu.VMEM((SEQ_LEN, HIDDEN), jnp.float32),
            [pltpu.SemaphoreType.DMA] * NBUF,
            [pltpu.SemaphoreType.DMA] * NBUF,
        ],
        compiler_params=pltpu.CompilerParams(use_tc_tiling_on_sc=False),
    )
    def k(idx_hbm, table_hbm, pos_hbm, out_hbm, idx_v, bufs, pos_v, gsems, ssems):
        wid = lax.axis_index("s") * NC + lax.axis_index("c")
        pltpu.sync_copy(pos_hbm, pos_v)
        pltpu.sync_copy(idx_hbm.at[wid], idx_v)

        def issue_gather(i, k_static):
            buf = bufs.at[k_static]
            pltpu.async_copy(table_hbm.at[idx_v.at[i, 0]],
                             buf.at[pl.ds(0, HALF)], gsems[k_static])
            pltpu.async_copy(table_hbm.at[idx_v.at[i, 1]],
                             buf.at[pl.ds(HALF, HALF)], gsems[k_static])

        def wait_gather(k_static):
            buf = bufs.at[k_static]
            dummy = table_hbm.at[pl.ds(0, HALF)]
            pltpu.make_async_copy(dummy, buf.at[pl.ds(0, HALF)],
                                  gsems[k_static]).wait()
            pltpu.make_async_copy(dummy, buf.at[pl.ds(HALF, HALF)],
                                  gsems[k_static]).wait()

        def wait_store(k_static):
            pltpu.make_async_copy(bufs.at[k_static], out_hbm.at[0],
                                  ssems[k_static]).wait()

        issue_gather(0, 0)
        issue_gather(1, 1)

        @pl.loop(0, BPW // NBUF)
        def _grp(j):
            for kk in range(NBUF):
                i = j * NBUF + kk
                buf = bufs.at[kk]
                wait_gather(kk)
                k2 = (kk + 2) % NBUF
                if kk < 2:
                    @pl.when(j > 0)
                    def _():
                        wait_store(k2)
                else:
                    wait_store(k2)
                if kk < 2:
                    issue_gather(i + 2, k2)
                else:
                    @pl.when(j < BPW // NBUF - 1)
                    def _():
                        issue_gather(i + 2, k2)

                @pl.loop(0, SEQ_LEN)
                def _row(r):
                    for d in range(HIDDEN // LANES):
                        sl = pl.ds(d * LANES, LANES)
                        buf[r, sl] = buf[r, sl] + pos_v[r, sl]

                pltpu.async_copy(buf, out_hbm.at[wid * BPW + i], ssems[kk])

        wait_store((BPW - 2) % NBUF)
        wait_store((BPW - 1) % NBUF)

    return k


_KERNEL = _build()


def kernel(item_seq, ID_embeddings, positional_embeddings):
    idx = item_seq.reshape(NW, BPW, 2, HALF)
    return _KERNEL(idx, ID_embeddings, positional_embeddings)
